# Initial kernel scaffold; baseline (speedup 1.0000x reference)
import functools

import jax
import jax.numpy as jnp
from jax import lax
from jax.experimental import pallas as pl
from jax.experimental.pallas import tpu as pltpu
from jax.experimental.pallas import tpu_sc as plsc

_N = 10000
_E = 320000
_B = 64
_NS = 16
_NC = 2
_CH = 80
_NCH = _E // _NS // _CH
_ZR = _N // _NS

_BN = 1000
_NG = _N // _BN


def _make_sc_agg(half):
  mesh = plsc.VectorSubcoreMesh(core_axis_name="c", subcore_axis_name="s")

  @functools.partial(
      pl.kernel,
      out_type=jax.ShapeDtypeStruct((_NC, _N, half), jnp.float32),
      mesh=mesh,
      scratch_types=[
          pltpu.VMEM((2, _NCH, _CH), jnp.int32),
          pltpu.VMEM((_CH, half), jnp.float32),
          pltpu.VMEM_SHARED((_N, half), jnp.float32),
          pltpu.SemaphoreType.DMA,
      ],
  )
  def sc_agg(h_hbm, src_hbm, dst_hbm, zero_hbm, out_hbm, idx_v, rows_v,
             acc_sh, sem):
    c = lax.axis_index("c")
    s = lax.axis_index("s")
    pltpu.sync_copy(src_hbm.at[s], idx_v.at[0])
    pltpu.sync_copy(dst_hbm.at[s], idx_v.at[1])
    pltpu.sync_copy(zero_hbm, acc_sh.at[pl.ds(s * _ZR, _ZR)])
    plsc.subcore_barrier()

    def body(j, carry):
      pltpu.async_copy(h_hbm.at[c].at[idx_v.at[0, j]], rows_v, sem).wait()
      pltpu.sync_copy(rows_v, acc_sh.at[idx_v.at[1, j]], add=True)
      return carry

    lax.fori_loop(0, _NCH, body, 0, unroll=False)

    plsc.subcore_barrier()
    pltpu.sync_copy(acc_sh.at[pl.ds(s * _ZR, _ZR)],
                    out_hbm.at[c].at[pl.ds(s * _ZR, _ZR)])

  return sc_agg


_sc_agg64 = _make_sc_agg(64)
_sc_agg128 = _make_sc_agg(128)
_sc_agg16 = _make_sc_agg(16)


def _dot(a, b):
  return jnp.dot(a, b, preferred_element_type=jnp.float32)


def _tc_layer_body(h_ref, a_ref, wrel_ref, wroot_ref, out_ref, *, hout):
  res = (_dot(a_ref[0], wrel_ref[0]) + _dot(a_ref[1], wrel_ref[1])
         + _dot(h_ref[0], wroot_ref[0]) + _dot(h_ref[1], wroot_ref[1]))
  res = jnp.maximum(res, 0.0)
  out_ref[0] = res[:, :hout]
  out_ref[1] = res[:, hout:]
  return res


def _tc_layer0(x_sp, agg0, w_rel, w_root):
  hin, cout = 64, 256
  body = functools.partial(_tc_layer_body, hout=cout // 2)
  return pl.pallas_call(
      body,
      grid=(_NG,),
      in_specs=[
          pl.BlockSpec((2, _BN, hin), lambda i: (0, i, 0)),
          pl.BlockSpec((2, _BN, hin), lambda i: (0, i, 0)),
          pl.BlockSpec((2, hin, cout), lambda i: (0, 0, 0)),
          pl.BlockSpec((2, hin, cout), lambda i: (0, 0, 0)),
      ],
      out_specs=pl.BlockSpec((2, _BN, cout // 2), lambda i: (0, i, 0)),
      out_shape=jax.ShapeDtypeStruct((2, _N, cout // 2), jnp.float32),
  )(x_sp, agg0, w_rel, w_root)


def _tc_layer1_body(h_ref, a_ref, wrel_ref, wroot_ref, wrel2_ref,
                    out_ref, p_ref):
  res = _tc_layer_body(h_ref, a_ref, wrel_ref, wroot_ref, out_ref, hout=128)
  p = _dot(res, wrel2_ref[...])
  p_ref[0] = p[:, :16]
  p_ref[1] = p[:, 16:]


def _tc_layer1(h1, agg1, w_rel, w_root, w_rel2):
  hin, cout = 128, 256
  return pl.pallas_call(
      _tc_layer1_body,
      grid=(_NG,),
      in_specs=[
          pl.BlockSpec((2, _BN, hin), lambda i: (0, i, 0)),
          pl.BlockSpec((2, _BN, hin), lambda i: (0, i, 0)),
          pl.BlockSpec((2, hin, cout), lambda i: (0, 0, 0)),
          pl.BlockSpec((2, hin, cout), lambda i: (0, 0, 0)),
          pl.BlockSpec((256, 32), lambda i: (0, 0)),
      ],
      out_specs=[
          pl.BlockSpec((2, _BN, cout // 2), lambda i: (0, i, 0)),
          pl.BlockSpec((2, _BN, 16), lambda i: (0, i, 0)),
      ],
      out_shape=[
          jax.ShapeDtypeStruct((2, _N, cout // 2), jnp.float32),
          jax.ShapeDtypeStruct((2, _N, 16), jnp.float32),
      ],
  )(h1, agg1, w_rel, w_root, w_rel2)


def _tc_final_body(h_ref, a_ref, wroot_ref, batch_ref, wfc1_ref, bfc1_ref,
                   wfc2_ref, bfc2_ref, out_ref, acc):
  i = pl.program_id(0)

  @pl.when(i == 0)
  def _zero():
    acc[...] = jnp.zeros_like(acc)

  a = jnp.concatenate([a_ref[0], a_ref[1]], axis=1)
  h3 = jnp.maximum(
      a + _dot(h_ref[0], wroot_ref[0]) + _dot(h_ref[1], wroot_ref[1]), 0.0)
  b = batch_ref[0]
  oh = (lax.broadcasted_iota(jnp.int32, (_B, _BN), 0) == b
        ).astype(jnp.float32)
  acc[...] += _dot(oh, h3)

  @pl.when(i == _NG - 1)
  def _head():
    hfc = jnp.maximum(_dot(acc[...], wfc1_ref[...]) + bfc1_ref[...], 0.0)
    out_ref[...] = _dot(hfc, wfc2_ref[...]) + bfc2_ref[...]


def _tc_final(h2, agg2, w_root2, batch3, wfc1, bfc1, wfc2, bfc2):
  return pl.pallas_call(
      _tc_final_body,
      grid=(_NG,),
      in_specs=[
          pl.BlockSpec((2, _BN, 128), lambda i: (0, i, 0)),
          pl.BlockSpec((2, _BN, 16), lambda i: (0, i, 0)),
          pl.BlockSpec((2, 128, 32), lambda i: (0, 0, 0)),
          pl.BlockSpec((1, 1, _BN), lambda i: (i, 0, 0)),
          pl.BlockSpec((32, 16), lambda i: (0, 0)),
          pl.BlockSpec((1, 16), lambda i: (0, 0)),
          pl.BlockSpec((16, 1), lambda i: (0, 0)),
          pl.BlockSpec((1, 1), lambda i: (0, 0)),
      ],
      out_specs=pl.BlockSpec((_B, 1), lambda i: (0, 0)),
      out_shape=jax.ShapeDtypeStruct((_B, 1), jnp.float32),
      scratch_shapes=[pltpu.VMEM((_B, 32), jnp.float32)],
  )(h2, agg2, w_root2, batch3, wfc1, bfc1, wfc2, bfc2)


def kernel(x, edge_index, batch, W_rel0, W_root0, W_rel1, W_root1, W_rel2,
           W_root2, W_fc11, b_fc11, W_fc12, b_fc12):
  src = edge_index[0].reshape(_NS, _NCH, _CH)
  dst = edge_index[1].reshape(_NS, _NCH, _CH)
  x_sp = jnp.moveaxis(x.reshape(_N, 2, 64), 1, 0)

  z64 = jnp.zeros((_ZR, 64), jnp.float32)
  z128 = jnp.zeros((_ZR, 128), jnp.float32)
  z16 = jnp.zeros((_ZR, 16), jnp.float32)

  agg0 = _sc_agg64(x_sp, src, dst, z64)
  h1 = _tc_layer0(x_sp, agg0, W_rel0.reshape(2, 64, 256),
                  W_root0.reshape(2, 64, 256))
  agg1 = _sc_agg128(h1, src, dst, z128)
  h2, p = _tc_layer1(h1, agg1, W_rel1.reshape(2, 128, 256),
                     W_root1.reshape(2, 128, 256), W_rel2)
  agg2 = _sc_agg16(p, src, dst, z16)
  batch3 = batch.reshape(_NG, 1, _BN)
  out = _tc_final(h2, agg2, W_root2.reshape(2, 128, 32), batch3,
                  W_fc11, b_fc11.reshape(1, 16), W_fc12, b_fc12.reshape(1, 1))
  return out


# speedup vs baseline: 6.4484x; 6.4484x over previous
import functools

import jax
import jax.numpy as jnp
from jax import lax
from jax.experimental import pallas as pl
from jax.experimental.pallas import tpu as pltpu
from jax.experimental.pallas import tpu_sc as plsc

_N = 10000
_E = 320000
_B = 64
_NS = 16
_NC = 2
_CH = 80
_NP = 10240
_ZR = _NP // _NS
_W = 128

_BN = 1000
_NG = _N // _BN

_mesh = plsc.VectorSubcoreMesh(core_axis_name="c", subcore_axis_name="s")


_NBLK = 125


def _sc_agg_common(h_view, src_blocks, dst_blocks, zero_hbm, out_view,
                   idx_v, rows_v, acc_sh, sem, s):
  pltpu.sync_copy(zero_hbm, acc_sh.at[pl.ds(s * _ZR, _ZR)])
  plsc.subcore_barrier()

  def body(j, carry):
    pltpu.async_copy(h_view.at[idx_v.at[0, j]], rows_v, sem).wait()
    pltpu.sync_copy(rows_v, acc_sh.at[idx_v.at[1, j]], add=True)
    return carry

  for src_t, dst_t in zip(src_blocks, dst_blocks):
    pltpu.sync_copy(src_t, idx_v.at[0])
    pltpu.sync_copy(dst_t, idx_v.at[1])
    lax.fori_loop(0, _NBLK, body, 0, unroll=False)

  plsc.subcore_barrier()
  pltpu.sync_copy(acc_sh.at[pl.ds(s * _ZR, _ZR)],
                  out_view.at[pl.ds(s * _ZR, _ZR)])


def _make_sc_agg_edge():
  @functools.partial(
      pl.kernel,
      out_type=jax.ShapeDtypeStruct((_NC, _NP, _W), jnp.float32),
      mesh=_mesh,
      scratch_types=[
          pltpu.VMEM((2, _NBLK, _CH), jnp.int32),
          pltpu.VMEM((_CH, _W), jnp.float32),
          pltpu.VMEM_SHARED((_NP, _W), jnp.float32),
          pltpu.SemaphoreType.DMA,
      ],
  )
  def sc_agg(h_hbm, src_hbm, dst_hbm, zero_hbm, out_hbm, idx_v, rows_v,
             acc_sh, sem):
    c = lax.axis_index("c")
    s = lax.axis_index("s")
    _sc_agg_common(h_hbm, [src_hbm.at[c, s]], [dst_hbm.at[c, s]], zero_hbm,
                   out_hbm.at[c], idx_v, rows_v, acc_sh, sem, s)

  return sc_agg


def _make_sc_agg_col():
  @functools.partial(
      pl.kernel,
      out_type=jax.ShapeDtypeStruct((_NC, _NP, _W), jnp.float32),
      mesh=_mesh,
      scratch_types=[
          pltpu.VMEM((2, _NBLK, _CH), jnp.int32),
          pltpu.VMEM((_CH, _W), jnp.float32),
          pltpu.VMEM_SHARED((_NP, _W), jnp.float32),
          pltpu.SemaphoreType.DMA,
      ],
  )
  def sc_agg(h_hbm, src_hbm, dst_hbm, zero_hbm, out_hbm, idx_v, rows_v,
             acc_sh, sem):
    c = lax.axis_index("c")
    s = lax.axis_index("s")
    _sc_agg_common(h_hbm.at[c], [src_hbm.at[s, 0], src_hbm.at[s, 1]],
                   [dst_hbm.at[s, 0], dst_hbm.at[s, 1]], zero_hbm,
                   out_hbm.at[c], idx_v, rows_v, acc_sh, sem, s)

  return sc_agg


_sc_agg_edge = _make_sc_agg_edge()
_sc_agg_col = _make_sc_agg_col()


def _dot(a, b):
  return jnp.dot(a, b, preferred_element_type=jnp.float32)


def _tc_layer0_body(x_ref, a_ref, wrel_ref, wroot_ref, out_ref):
  agg = a_ref[0] + a_ref[1]
  res = jnp.maximum(_dot(agg, wrel_ref[...]) + _dot(x_ref[...], wroot_ref[...]),
                    0.0)
  out_ref[0] = res[:, :128]
  out_ref[1] = res[:, 128:]


def _tc_layer0(x, agg0, w_rel, w_root):
  return pl.pallas_call(
      _tc_layer0_body,
      grid=(_NG,),
      in_specs=[
          pl.BlockSpec((_BN, 128), lambda i: (i, 0)),
          pl.BlockSpec((2, _BN, 128), lambda i: (0, i, 0)),
          pl.BlockSpec((128, 256), lambda i: (0, 0)),
          pl.BlockSpec((128, 256), lambda i: (0, 0)),
      ],
      out_specs=pl.BlockSpec((2, _BN, 128), lambda i: (0, i, 0)),
      out_shape=jax.ShapeDtypeStruct((2, _N, 128), jnp.float32),
  )(x, agg0, w_rel, w_root)


def _tc_layer1_body(h_ref, a_ref, wrel_ref, wroot_ref, wrel2_ref,
                    out_ref, p_ref):
  res = (_dot(a_ref[0], wrel_ref[0]) + _dot(a_ref[1], wrel_ref[1])
         + _dot(h_ref[0], wroot_ref[0]) + _dot(h_ref[1], wroot_ref[1]))
  res = jnp.maximum(res, 0.0)
  out_ref[0] = res[:, :128]
  out_ref[1] = res[:, 128:]
  p = _dot(res, wrel2_ref[...])
  p_ref[...] = jnp.pad(p, ((0, 0), (0, 96)))


def _tc_layer1(h1, agg1, w_rel, w_root, w_rel2):
  return pl.pallas_call(
      _tc_layer1_body,
      grid=(_NG,),
      in_specs=[
          pl.BlockSpec((2, _BN, 128), lambda i: (0, i, 0)),
          pl.BlockSpec((2, _BN, 128), lambda i: (0, i, 0)),
          pl.BlockSpec((2, 128, 256), lambda i: (0, 0, 0)),
          pl.BlockSpec((2, 128, 256), lambda i: (0, 0, 0)),
          pl.BlockSpec((256, 32), lambda i: (0, 0)),
      ],
      out_specs=[
          pl.BlockSpec((2, _BN, 128), lambda i: (0, i, 0)),
          pl.BlockSpec((_BN, 128), lambda i: (i, 0)),
      ],
      out_shape=[
          jax.ShapeDtypeStruct((2, _N, 128), jnp.float32),
          jax.ShapeDtypeStruct((_N, 128), jnp.float32),
      ],
  )(h1, agg1, w_rel, w_root, w_rel2)


def _tc_final_body(h_ref, a_ref, wroot_ref, batch_ref, wfc1_ref, bfc1_ref,
                   wfc2_ref, bfc2_ref, out_ref, acc):
  i = pl.program_id(0)

  @pl.when(i == 0)
  def _zero():
    acc[...] = jnp.zeros_like(acc)

  a = a_ref[0] + a_ref[1]
  h3 = jnp.maximum(
      a + _dot(h_ref[0], wroot_ref[0]) + _dot(h_ref[1], wroot_ref[1]), 0.0)
  b = batch_ref[0]
  oh = (lax.broadcasted_iota(jnp.int32, (_B, _BN), 0) == b
        ).astype(jnp.float32)
  acc[...] += _dot(oh, h3)

  @pl.when(i == _NG - 1)
  def _head():
    hfc = jnp.maximum(_dot(acc[...], wfc1_ref[...]) + bfc1_ref[...], 0.0)
    out_ref[...] = _dot(hfc, wfc2_ref[...]) + bfc2_ref[...]


def _tc_final(h2, agg2, w_root2, batch3, wfc1, bfc1, wfc2, bfc2):
  return pl.pallas_call(
      _tc_final_body,
      grid=(_NG,),
      in_specs=[
          pl.BlockSpec((2, _BN, 128), lambda i: (0, i, 0)),
          pl.BlockSpec((2, _BN, 32), lambda i: (0, i, 0)),
          pl.BlockSpec((2, 128, 32), lambda i: (0, 0, 0)),
          pl.BlockSpec((1, 1, _BN), lambda i: (i, 0, 0)),
          pl.BlockSpec((32, 16), lambda i: (0, 0)),
          pl.BlockSpec((1, 16), lambda i: (0, 0)),
          pl.BlockSpec((16, 1), lambda i: (0, 0)),
          pl.BlockSpec((1, 1), lambda i: (0, 0)),
      ],
      out_specs=pl.BlockSpec((_B, 1), lambda i: (0, 0)),
      out_shape=jax.ShapeDtypeStruct((_B, 1), jnp.float32),
      scratch_shapes=[pltpu.VMEM((_B, 32), jnp.float32)],
  )(h2, agg2, w_root2, batch3, wfc1, bfc1, wfc2, bfc2)


def kernel(x, edge_index, batch, W_rel0, W_root0, W_rel1, W_root1, W_rel2,
           W_root2, W_fc11, b_fc11, W_fc12, b_fc12):
  src_e = edge_index[0].reshape(_NC, _NS, _NBLK, _CH)
  dst_e = edge_index[1].reshape(_NC, _NS, _NBLK, _CH)
  src_c = edge_index[0].reshape(_NS, 2, _NBLK, _CH)
  dst_c = edge_index[1].reshape(_NS, 2, _NBLK, _CH)
  z128 = jnp.zeros((_ZR, _W), jnp.float32)

  agg0 = _sc_agg_edge(x, src_e, dst_e, z128)
  h1 = _tc_layer0(x, agg0, W_rel0, W_root0)
  agg1 = _sc_agg_col(h1, src_c, dst_c, z128)
  h2, p = _tc_layer1(h1, agg1, W_rel1.reshape(2, 128, 256),
                     W_root1.reshape(2, 128, 256), W_rel2)
  agg2 = _sc_agg_edge(p, src_e, dst_e, z128)
  batch3 = batch.reshape(_NG, 1, _BN)
  out = _tc_final(h2, agg2[:, :, :32], W_root2.reshape(2, 128, 32), batch3,
                  W_fc11, b_fc11.reshape(1, 16), W_fc12, b_fc12.reshape(1, 1))
  return out


# speedup vs baseline: 8.0183x; 1.2435x over previous
import functools

import jax
import jax.numpy as jnp
from jax import lax
from jax.experimental import pallas as pl
from jax.experimental.pallas import tpu as pltpu
from jax.experimental.pallas import tpu_sc as plsc

_N = 10000
_E = 320000
_B = 64
_NS = 16
_NC = 2
_CH = 80
_NP = 10240
_ZR = _NP // _NS
_W = 128

_BN = 1000
_NG = _N // _BN

_mesh = plsc.VectorSubcoreMesh(core_axis_name="c", subcore_axis_name="s")


_NBLK = 25


def _sc_agg_common(h_view, src_blocks, dst_blocks, zero_hbm, out_view,
                   idx_v, rows_v, acc_sh, sems, s):
  pltpu.sync_copy(zero_hbm, acc_sh.at[pl.ds(s * _ZR, _ZR)])
  plsc.subcore_barrier()

  nblocks = len(src_blocks)
  for bi, (src_t, dst_t) in enumerate(zip(src_blocks, dst_blocks)):
    pltpu.sync_copy(src_t, idx_v.at[0])
    pltpu.sync_copy(dst_t, idx_v.at[1])
    pltpu.async_copy(h_view.at[idx_v.at[0, 0]], rows_v.at[0], sems.at[0])

    def body(j, carry):
      b = lax.rem(j, 2)
      pltpu.make_async_copy(
          h_view.at[idx_v.at[0, j]], rows_v.at[b], sems.at[b]).wait()

      @pl.when(j + 1 < _NBLK)
      def _next():
        pltpu.async_copy(
            h_view.at[idx_v.at[0, j + 1]], rows_v.at[1 - b], sems.at[1 - b])

      pltpu.sync_copy(rows_v.at[b], acc_sh.at[idx_v.at[1, j]], add=True)
      return carry

    lax.fori_loop(0, _NBLK, body, 0, unroll=False)

  plsc.subcore_barrier()
  pltpu.sync_copy(acc_sh.at[pl.ds(s * _ZR, _ZR)],
                  out_view.at[pl.ds(s * _ZR, _ZR)])


def _make_sc_agg_edge():
  @functools.partial(
      pl.kernel,
      out_type=jax.ShapeDtypeStruct((_NC, _NP, _W), jnp.float32),
      mesh=_mesh,
      scratch_types=[
          pltpu.VMEM((2, _NBLK, _CH), jnp.int32),
          pltpu.VMEM((2, _CH, _W), jnp.float32),
          pltpu.VMEM_SHARED((_NP, _W), jnp.float32),
          pltpu.SemaphoreType.DMA((2,)),
      ],
  )
  def sc_agg(h_hbm, src_hbm, dst_hbm, zero_hbm, out_hbm, idx_v, rows_v,
             acc_sh, sems):
    c = lax.axis_index("c")
    s = lax.axis_index("s")
    _sc_agg_common(h_hbm, [src_hbm.at[c, s, b] for b in range(5)],
                   [dst_hbm.at[c, s, b] for b in range(5)], zero_hbm,
                   out_hbm.at[c], idx_v, rows_v, acc_sh, sems, s)

  return sc_agg


def _make_sc_agg_col():
  @functools.partial(
      pl.kernel,
      out_type=jax.ShapeDtypeStruct((_NC, _NP, _W), jnp.float32),
      mesh=_mesh,
      scratch_types=[
          pltpu.VMEM((2, _NBLK, _CH), jnp.int32),
          pltpu.VMEM((2, _CH, _W), jnp.float32),
          pltpu.VMEM_SHARED((_NP, _W), jnp.float32),
          pltpu.SemaphoreType.DMA((2,)),
      ],
  )
  def sc_agg(h_hbm, src_hbm, dst_hbm, zero_hbm, out_hbm, idx_v, rows_v,
             acc_sh, sems):
    c = lax.axis_index("c")
    s = lax.axis_index("s")
    _sc_agg_common(h_hbm.at[c], [src_hbm.at[s, b] for b in range(10)],
                   [dst_hbm.at[s, b] for b in range(10)], zero_hbm,
                   out_hbm.at[c], idx_v, rows_v, acc_sh, sems, s)

  return sc_agg


_sc_agg_edge = _make_sc_agg_edge()
_sc_agg_col = _make_sc_agg_col()


def _dot(a, b):
  return jnp.dot(a, b, preferred_element_type=jnp.float32)


def _tc_layer0_body(x_ref, a_ref, wrel_ref, wroot_ref, out_ref):
  agg = a_ref[0] + a_ref[1]
  res = jnp.maximum(_dot(agg, wrel_ref[...]) + _dot(x_ref[...], wroot_ref[...]),
                    0.0)
  out_ref[0] = res[:, :128]
  out_ref[1] = res[:, 128:]


def _tc_layer0(x, agg0, w_rel, w_root):
  return pl.pallas_call(
      _tc_layer0_body,
      grid=(_NG,),
      in_specs=[
          pl.BlockSpec((_BN, 128), lambda i: (i, 0)),
          pl.BlockSpec((2, _BN, 128), lambda i: (0, i, 0)),
          pl.BlockSpec((128, 256), lambda i: (0, 0)),
          pl.BlockSpec((128, 256), lambda i: (0, 0)),
      ],
      out_specs=pl.BlockSpec((2, _BN, 128), lambda i: (0, i, 0)),
      out_shape=jax.ShapeDtypeStruct((2, _N, 128), jnp.float32),
  )(x, agg0, w_rel, w_root)


def _tc_layer1_body(h_ref, a_ref, wrel_ref, wroot_ref, wrel2_ref,
                    out_ref, p_ref):
  res = (_dot(a_ref[0], wrel_ref[0]) + _dot(a_ref[1], wrel_ref[1])
         + _dot(h_ref[0], wroot_ref[0]) + _dot(h_ref[1], wroot_ref[1]))
  res = jnp.maximum(res, 0.0)
  out_ref[0] = res[:, :128]
  out_ref[1] = res[:, 128:]
  p = _dot(res, wrel2_ref[...])
  p_ref[...] = jnp.pad(p, ((0, 0), (0, 96)))


def _tc_layer1(h1, agg1, w_rel, w_root, w_rel2):
  return pl.pallas_call(
      _tc_layer1_body,
      grid=(_NG,),
      in_specs=[
          pl.BlockSpec((2, _BN, 128), lambda i: (0, i, 0)),
          pl.BlockSpec((2, _BN, 128), lambda i: (0, i, 0)),
          pl.BlockSpec((2, 128, 256), lambda i: (0, 0, 0)),
          pl.BlockSpec((2, 128, 256), lambda i: (0, 0, 0)),
          pl.BlockSpec((256, 32), lambda i: (0, 0)),
      ],
      out_specs=[
          pl.BlockSpec((2, _BN, 128), lambda i: (0, i, 0)),
          pl.BlockSpec((_BN, 128), lambda i: (i, 0)),
      ],
      out_shape=[
          jax.ShapeDtypeStruct((2, _N, 128), jnp.float32),
          jax.ShapeDtypeStruct((_N, 128), jnp.float32),
      ],
  )(h1, agg1, w_rel, w_root, w_rel2)


def _tc_final_body(h_ref, a_ref, wroot_ref, batch_ref, wfc1_ref, bfc1_ref,
                   wfc2_ref, bfc2_ref, out_ref, acc):
  i = pl.program_id(0)

  @pl.when(i == 0)
  def _zero():
    acc[...] = jnp.zeros_like(acc)

  a = a_ref[0] + a_ref[1]
  h3 = jnp.maximum(
      a + _dot(h_ref[0], wroot_ref[0]) + _dot(h_ref[1], wroot_ref[1]), 0.0)
  b = batch_ref[0]
  oh = (lax.broadcasted_iota(jnp.int32, (_B, _BN), 0) == b
        ).astype(jnp.float32)
  acc[...] += _dot(oh, h3)

  @pl.when(i == _NG - 1)
  def _head():
    hfc = jnp.maximum(_dot(acc[...], wfc1_ref[...]) + bfc1_ref[...], 0.0)
    out_ref[...] = _dot(hfc, wfc2_ref[...]) + bfc2_ref[...]


def _tc_final(h2, agg2, w_root2, batch3, wfc1, bfc1, wfc2, bfc2):
  return pl.pallas_call(
      _tc_final_body,
      grid=(_NG,),
      in_specs=[
          pl.BlockSpec((2, _BN, 128), lambda i: (0, i, 0)),
          pl.BlockSpec((2, _BN, 32), lambda i: (0, i, 0)),
          pl.BlockSpec((2, 128, 32), lambda i: (0, 0, 0)),
          pl.BlockSpec((1, 1, _BN), lambda i: (i, 0, 0)),
          pl.BlockSpec((32, 16), lambda i: (0, 0)),
          pl.BlockSpec((1, 16), lambda i: (0, 0)),
          pl.BlockSpec((16, 1), lambda i: (0, 0)),
          pl.BlockSpec((1, 1), lambda i: (0, 0)),
      ],
      out_specs=pl.BlockSpec((_B, 1), lambda i: (0, 0)),
      out_shape=jax.ShapeDtypeStruct((_B, 1), jnp.float32),
      scratch_shapes=[pltpu.VMEM((_B, 32), jnp.float32)],
  )(h2, agg2, w_root2, batch3, wfc1, bfc1, wfc2, bfc2)


def kernel(x, edge_index, batch, W_rel0, W_root0, W_rel1, W_root1, W_rel2,
           W_root2, W_fc11, b_fc11, W_fc12, b_fc12):
  src_e = edge_index[0].reshape(_NC, _NS, 5, _NBLK, _CH)
  dst_e = edge_index[1].reshape(_NC, _NS, 5, _NBLK, _CH)
  src_c = edge_index[0].reshape(_NS, 10, _NBLK, _CH)
  dst_c = edge_index[1].reshape(_NS, 10, _NBLK, _CH)
  z128 = jnp.zeros((_ZR, _W), jnp.float32)

  agg0 = _sc_agg_edge(x, src_e, dst_e, z128)
  h1 = _tc_layer0(x, agg0, W_rel0, W_root0)
  agg1 = _sc_agg_col(h1, src_c, dst_c, z128)
  h2, p = _tc_layer1(h1, agg1, W_rel1.reshape(2, 128, 256),
                     W_root1.reshape(2, 128, 256), W_rel2)
  agg2 = _sc_agg_edge(p, src_e, dst_e, z128)
  batch3 = batch.reshape(_NG, 1, _BN)
  out = _tc_final(h2, agg2[:, :, :32], W_root2.reshape(2, 128, 32), batch3,
                  W_fc11, b_fc11.reshape(1, 16), W_fc12, b_fc12.reshape(1, 1))
  return out


# speedup vs baseline: 9.4321x; 1.1763x over previous
import functools

import jax
import jax.numpy as jnp
from jax import lax
from jax.experimental import pallas as pl
from jax.experimental.pallas import tpu as pltpu
from jax.experimental.pallas import tpu_sc as plsc

_N = 10000
_E = 320000
_B = 64
_NS = 16
_NC = 2
_CH = 128
_NP = 10240
_ZR = _NP // _NS
_W = 128

_BN = 1000
_NG = _N // _BN

_mesh = plsc.VectorSubcoreMesh(core_axis_name="c", subcore_axis_name="s")


_NBLK = 20


def _sc_agg_common(h_view, src_blocks, dst_blocks, zero_hbm, out_view,
                   idx_v, rows_v, acc_sh, gsems, s, scat_w=_W):
  pltpu.sync_copy(zero_hbm, acc_sh.at[pl.ds(s * _ZR, _ZR)])
  plsc.subcore_barrier()

  if scat_w == _W:
    scat_src = lambda b: rows_v.at[b]
  else:
    scat_src = lambda b: rows_v.at[b].at[:, pl.ds(0, scat_w)]

  for src_t, dst_t in zip(src_blocks, dst_blocks):
    pltpu.sync_copy(src_t, idx_v.at[0])
    pltpu.sync_copy(dst_t, idx_v.at[1])
    pltpu.async_copy(h_view.at[idx_v.at[0, 0]], rows_v.at[0], gsems.at[0])

    def body(j, carry):
      b = lax.rem(j, 2)
      pltpu.make_async_copy(
          h_view.at[idx_v.at[0, j]], rows_v.at[b], gsems.at[b]).wait()

      @pl.when(j + 1 < _NBLK)
      def _next():
        pltpu.async_copy(
            h_view.at[idx_v.at[0, j + 1]], rows_v.at[1 - b], gsems.at[1 - b])

      pltpu.sync_copy(scat_src(b), acc_sh.at[idx_v.at[1, j]], add=True)
      return carry

    lax.fori_loop(0, _NBLK, body, 0, unroll=False)

  plsc.subcore_barrier()
  pltpu.sync_copy(acc_sh.at[pl.ds(s * _ZR, _ZR)],
                  out_view.at[pl.ds(s * _ZR, _ZR)])


def _make_sc_agg_edge(acc_w):
  @functools.partial(
      pl.kernel,
      out_type=jax.ShapeDtypeStruct((_NC, _NP, acc_w), jnp.float32),
      mesh=_mesh,
      scratch_types=[
          pltpu.VMEM((2, _NBLK, _CH), jnp.int32),
          pltpu.VMEM((2, _CH, _W), jnp.float32),
          pltpu.VMEM_SHARED((_NP, acc_w), jnp.float32),
          pltpu.SemaphoreType.DMA((2,)),
      ],
  )
  def sc_agg(h_hbm, src_hbm, dst_hbm, zero_hbm, out_hbm, idx_v, rows_v,
             acc_sh, gsems):
    c = lax.axis_index("c")
    s = lax.axis_index("s")
    _sc_agg_common(h_hbm, [src_hbm.at[c, s, b] for b in range(4)],
                   [dst_hbm.at[c, s, b] for b in range(4)], zero_hbm,
                   out_hbm.at[c], idx_v, rows_v, acc_sh, gsems, s,
                   scat_w=acc_w)

  return sc_agg


def _make_sc_agg_col():
  @functools.partial(
      pl.kernel,
      out_type=jax.ShapeDtypeStruct((_NC, _NP, _W), jnp.float32),
      mesh=_mesh,
      scratch_types=[
          pltpu.VMEM((2, _NBLK, _CH), jnp.int32),
          pltpu.VMEM((2, _CH, _W), jnp.float32),
          pltpu.VMEM_SHARED((_NP, _W), jnp.float32),
          pltpu.SemaphoreType.DMA((2,)),
      ],
  )
  def sc_agg(h_hbm, src_hbm, dst_hbm, zero_hbm, out_hbm, idx_v, rows_v,
             acc_sh, gsems):
    c = lax.axis_index("c")
    s = lax.axis_index("s")
    _sc_agg_common(h_hbm.at[c], [src_hbm.at[s, b] for b in range(8)],
                   [dst_hbm.at[s, b] for b in range(8)], zero_hbm,
                   out_hbm.at[c], idx_v, rows_v, acc_sh, gsems, s)

  return sc_agg


_sc_agg_edge128 = _make_sc_agg_edge(128)
_sc_agg_edge32 = _make_sc_agg_edge(128)
_sc_agg_col = _make_sc_agg_col()


def _dot(a, b):
  return jnp.dot(a, b, preferred_element_type=jnp.float32)


def _tc_layer0_body(x_ref, a_ref, wrel_ref, wroot_ref, out_ref):
  agg = a_ref[0] + a_ref[1]
  res = jnp.maximum(_dot(agg, wrel_ref[...]) + _dot(x_ref[...], wroot_ref[...]),
                    0.0)
  out_ref[0] = res[:, :128]
  out_ref[1] = res[:, 128:]


def _tc_layer0(x, agg0, w_rel, w_root):
  return pl.pallas_call(
      _tc_layer0_body,
      grid=(_NG,),
      in_specs=[
          pl.BlockSpec((_BN, 128), lambda i: (i, 0)),
          pl.BlockSpec((2, _BN, 128), lambda i: (0, i, 0)),
          pl.BlockSpec((128, 256), lambda i: (0, 0)),
          pl.BlockSpec((128, 256), lambda i: (0, 0)),
      ],
      out_specs=pl.BlockSpec((2, _BN, 128), lambda i: (0, i, 0)),
      out_shape=jax.ShapeDtypeStruct((2, _N, 128), jnp.float32),
  )(x, agg0, w_rel, w_root)


def _tc_layer1_body(h_ref, a_ref, wrel_ref, wroot_ref, wrel2_ref,
                    out_ref, p_ref):
  res = (_dot(a_ref[0], wrel_ref[0]) + _dot(a_ref[1], wrel_ref[1])
         + _dot(h_ref[0], wroot_ref[0]) + _dot(h_ref[1], wroot_ref[1]))
  res = jnp.maximum(res, 0.0)
  out_ref[0] = res[:, :128]
  out_ref[1] = res[:, 128:]
  p = _dot(res, wrel2_ref[...])
  p_ref[...] = jnp.pad(p, ((0, 0), (0, 96)))


def _tc_layer1(h1, agg1, w_rel, w_root, w_rel2):
  return pl.pallas_call(
      _tc_layer1_body,
      grid=(_NG,),
      in_specs=[
          pl.BlockSpec((2, _BN, 128), lambda i: (0, i, 0)),
          pl.BlockSpec((2, _BN, 128), lambda i: (0, i, 0)),
          pl.BlockSpec((2, 128, 256), lambda i: (0, 0, 0)),
          pl.BlockSpec((2, 128, 256), lambda i: (0, 0, 0)),
          pl.BlockSpec((256, 32), lambda i: (0, 0)),
      ],
      out_specs=[
          pl.BlockSpec((2, _BN, 128), lambda i: (0, i, 0)),
          pl.BlockSpec((_BN, 128), lambda i: (i, 0)),
      ],
      out_shape=[
          jax.ShapeDtypeStruct((2, _N, 128), jnp.float32),
          jax.ShapeDtypeStruct((_N, 128), jnp.float32),
      ],
  )(h1, agg1, w_rel, w_root, w_rel2)


def _tc_final_body(h_ref, a_ref, wroot_ref, batch_ref, wfc1_ref, bfc1_ref,
                   wfc2_ref, bfc2_ref, out_ref, acc):
  i = pl.program_id(0)

  @pl.when(i == 0)
  def _zero():
    acc[...] = jnp.zeros_like(acc)

  a = a_ref[0] + a_ref[1]
  h3 = jnp.maximum(
      a + _dot(h_ref[0], wroot_ref[0]) + _dot(h_ref[1], wroot_ref[1]), 0.0)
  b = batch_ref[0]
  oh = (lax.broadcasted_iota(jnp.int32, (_B, _BN), 0) == b
        ).astype(jnp.float32)
  acc[...] += _dot(oh, h3)

  @pl.when(i == _NG - 1)
  def _head():
    hfc = jnp.maximum(_dot(acc[...], wfc1_ref[...]) + bfc1_ref[...], 0.0)
    out_ref[...] = _dot(hfc, wfc2_ref[...]) + bfc2_ref[...]


def _tc_final(h2, agg2, w_root2, batch3, wfc1, bfc1, wfc2, bfc2):
  return pl.pallas_call(
      _tc_final_body,
      grid=(_NG,),
      in_specs=[
          pl.BlockSpec((2, _BN, 128), lambda i: (0, i, 0)),
          pl.BlockSpec((2, _BN, 32), lambda i: (0, i, 0)),
          pl.BlockSpec((2, 128, 32), lambda i: (0, 0, 0)),
          pl.BlockSpec((1, 1, _BN), lambda i: (i, 0, 0)),
          pl.BlockSpec((32, 16), lambda i: (0, 0)),
          pl.BlockSpec((1, 16), lambda i: (0, 0)),
          pl.BlockSpec((16, 1), lambda i: (0, 0)),
          pl.BlockSpec((1, 1), lambda i: (0, 0)),
      ],
      out_specs=pl.BlockSpec((_B, 1), lambda i: (0, 0)),
      out_shape=jax.ShapeDtypeStruct((_B, 1), jnp.float32),
      scratch_shapes=[pltpu.VMEM((_B, 32), jnp.float32)],
  )(h2, agg2, w_root2, batch3, wfc1, bfc1, wfc2, bfc2)


def kernel(x, edge_index, batch, W_rel0, W_root0, W_rel1, W_root1, W_rel2,
           W_root2, W_fc11, b_fc11, W_fc12, b_fc12):
  def _pad_edges(arr, rows, fill_mod, fill_base):
    per = arr.shape[-1]
    pad = -per % (_NBLK * _CH)
    fill = (fill_base
            + (jnp.arange(rows * pad, dtype=jnp.int32) % fill_mod)
            ).reshape(rows, pad)
    return jnp.concatenate([arr, fill], axis=1)

  ept_e = _E // (_NC * _NS)
  ept_c = _E // _NS
  src_e = _pad_edges(edge_index[0].reshape(_NC * _NS, ept_e), _NC * _NS,
                     _N, 0).reshape(_NC, _NS, -1, _NBLK, _CH)
  dst_e = _pad_edges(edge_index[1].reshape(_NC * _NS, ept_e), _NC * _NS,
                     _NP - _N, _N).reshape(_NC, _NS, -1, _NBLK, _CH)
  src_c = _pad_edges(edge_index[0].reshape(_NS, ept_c), _NS,
                     _N, 0).reshape(_NS, -1, _NBLK, _CH)
  dst_c = _pad_edges(edge_index[1].reshape(_NS, ept_c), _NS,
                     _NP - _N, _N).reshape(_NS, -1, _NBLK, _CH)
  z128 = jnp.zeros((_ZR, _W), jnp.float32)
  z32 = jnp.zeros((_ZR, 32), jnp.float32)

  agg0 = _sc_agg_edge128(x, src_e, dst_e, z128)
  h1 = _tc_layer0(x, agg0, W_rel0, W_root0)
  agg1 = _sc_agg_col(h1, src_c, dst_c, z128)
  h2, p = _tc_layer1(h1, agg1, W_rel1.reshape(2, 128, 256),
                     W_root1.reshape(2, 128, 256), W_rel2)
  agg2 = _sc_agg_edge32(p, src_e, dst_e, z128)
  batch3 = batch.reshape(_NG, 1, _BN)
  out = _tc_final(h2, agg2[:, :, :32], W_root2.reshape(2, 128, 32), batch3,
                  W_fc11, b_fc11.reshape(1, 16), W_fc12, b_fc12.reshape(1, 1))
  return out
